# baseline (speedup 1.0000x reference)
import functools

import jax
import jax.numpy as jnp
from jax import lax
from jax.experimental import pallas as pl
from jax.experimental.pallas import tpu as pltpu
from jax.experimental.pallas import tpu_sc as plsc

B, S, DIM = 2, 2048, 768
E, K, HIDDEN = 8, 2, 2048
TOK = B * S
NA = TOK * K

RTB = 1024
RNB = TOK // RTB

BT = 512
NBG = NA // BT
TG = NBG + E - 1


def _router_body(x_ref, rw_ref, rb_ref,
                 d1_ref, d2_ref, w1x_ref, w2x_ref, tiles_ref,
                 acc_ref, se1_ref, se2_ref, sr1_ref, sr2_ref, sw1_ref):
    i = pl.program_id(0)

    @pl.when(i == 0)
    def _():
        acc_ref[...] = jnp.zeros_like(acc_ref)

    @pl.when(i < RNB)
    def _phase0():
        prev = acc_ref[...]
        x = x_ref[...]
        logits = jax.lax.dot_general(x, rw_ref[...], (((1,), (1,)), ((), ())),
                                     preferred_element_type=jnp.float32)
        logits = logits + rb_ref[...]

        eids = jax.lax.broadcasted_iota(jnp.int32, (RTB, E), 1)
        v1 = jnp.max(logits, axis=1, keepdims=True)
        i1 = jnp.min(jnp.where(logits == v1, eids, E), axis=1)
        masked = jnp.where(eids == i1[:, None], -jnp.inf, logits)
        v2 = jnp.max(masked, axis=1, keepdims=True)
        i2 = jnp.min(jnp.where(masked == v2, eids, E), axis=1)

        w1 = 1.0 / (1.0 + jnp.exp(v2[:, 0] - v1[:, 0]))

        a1 = (eids == i1[:, None]).astype(jnp.float32)
        a2 = (eids == i2[:, None]).astype(jnp.float32)
        s = a1 + a2
        tri = (jax.lax.broadcasted_iota(jnp.int32, (RTB, RTB), 0)
               > jax.lax.broadcasted_iota(jnp.int32, (RTB, RTB), 1)
               ).astype(jnp.float32)
        csum = jax.lax.dot_general(tri, s, (((1,), (0,)), ((), ())),
                                   precision=jax.lax.Precision.HIGHEST,
                                   preferred_element_type=jnp.float32)
        ctot = csum + prev
        r1 = jnp.sum(ctot * a1, axis=1)
        r2 = jnp.sum(ctot * a2, axis=1)

        se1_ref[pl.ds(i, 1), :] = i1[None, :]
        se2_ref[pl.ds(i, 1), :] = i2[None, :]
        sr1_ref[pl.ds(i, 1), :] = r1[None, :]
        sr2_ref[pl.ds(i, 1), :] = r2[None, :]
        sw1_ref[pl.ds(i * RTB, RTB), :] = w1[:, None]
        acc_ref[...] = prev + jnp.sum(s, axis=0, keepdims=True)

    @pl.when(i == RNB)
    def _finalize():
        def rowdot(a, b):
            return jax.lax.dot_general(a, b, (((1,), (0,)), ((), ())),
                                       precision=jax.lax.Precision.HIGHEST,
                                       preferred_element_type=jnp.float32)

        cnt = acc_ref[...]
        lt = (jax.lax.broadcasted_iota(jnp.int32, (E, E), 0)
              < jax.lax.broadcasted_iota(jnp.int32, (E, E), 1)).astype(jnp.float32)
        off_e = rowdot(cnt, lt)
        off_i = off_e + cnt

        for rb in range(RNB):
            e1b = se1_ref[pl.ds(rb, 1), :]
            e2b = se2_ref[pl.ds(rb, 1), :]
            eids = jax.lax.broadcasted_iota(jnp.int32, (E, RTB), 0)
            a1 = (eids == e1b).astype(jnp.float32)
            a2 = (eids == e2b).astype(jnp.float32)
            d1 = rowdot(off_e, a1) + sr1_ref[pl.ds(rb, 1), :]
            d2 = rowdot(off_e, a2) + sr2_ref[pl.ds(rb, 1), :]
            d1_ref[0, :, rb * RTB:(rb + 1) * RTB] = d1.astype(jnp.int32)
            d2_ref[0, :, rb * RTB:(rb + 1) * RTB] = d2.astype(jnp.int32)

        w1x_ref[...] = jnp.broadcast_to(sw1_ref[...], (TOK, 16))
        w2x_ref[...] = 1.0 - w1x_ref[...]

        inv_bt = 1.0 / BT
        nonempty = (cnt > 0).astype(jnp.float32)
        nb = (jnp.floor((off_i - 1.0) * inv_bt) - jnp.floor(off_e * inv_bt)
              + 1.0) * nonempty
        le = (jax.lax.broadcasted_iota(jnp.int32, (E, E), 0)
              <= jax.lax.broadcasted_iota(jnp.int32, (E, E), 1)).astype(jnp.float32)
        ts_i = rowdot(nb, le)
        ts_e = ts_i - nb
        total = jnp.max(ts_i, axis=1, keepdims=True)

        ident = (jax.lax.broadcasted_iota(jnp.int32, (E, E), 0)
                 == jax.lax.broadcasted_iota(jnp.int32, (E, E), 1)
                 ).astype(jnp.float32)
        ts_col = jax.lax.dot_general(ident, ts_i, (((1,), (1,)), ((), ())),
                                     precision=jax.lax.Precision.HIGHEST,
                                     preferred_element_type=jnp.float32)

        tids = jax.lax.broadcasted_iota(jnp.int32, (1, TG), 1).astype(jnp.float32)
        tc = jnp.minimum(tids, total - 1.0)
        cmp = (ts_col <= tc).astype(jnp.float32)
        et = jnp.sum(cmp, axis=0, keepdims=True)
        sel = (jax.lax.broadcasted_iota(jnp.int32, (E, TG), 0).astype(jnp.float32)
               == et).astype(jnp.float32)
        offe_t = rowdot(off_e, sel)
        offi_t = rowdot(off_i, sel)
        tse_t = rowdot(ts_e, sel)
        bt = jnp.floor(offe_t * inv_bt) + (tc - tse_t)
        lo = jnp.maximum(offe_t - bt * BT, 0.0)
        hi = jnp.minimum(offi_t - bt * BT, float(BT))
        tiles = jnp.concatenate([bt, et, lo, hi], axis=0)
        tiles_ref[...] = tiles.astype(jnp.int32)[None]


def _run_router(x2d, router_W, router_b):
    out_shapes = (
        jax.ShapeDtypeStruct((1, 1, TOK), jnp.int32),
        jax.ShapeDtypeStruct((1, 1, TOK), jnp.int32),
        jax.ShapeDtypeStruct((TOK, 16), jnp.float32),
        jax.ShapeDtypeStruct((TOK, 16), jnp.float32),
        jax.ShapeDtypeStruct((1, 4, TG), jnp.int32),
    )
    z3 = lambda i: (0, 0, 0)
    return pl.pallas_call(
        _router_body,
        grid=(RNB + 1,),
        in_specs=[
            pl.BlockSpec((RTB, DIM), lambda i: (jnp.minimum(i, RNB - 1), 0)),
            pl.BlockSpec((E, DIM), lambda i: (0, 0)),
            pl.BlockSpec((1, E), lambda i: (0, 0)),
        ],
        out_specs=(
            pl.BlockSpec((1, 1, TOK), z3), pl.BlockSpec((1, 1, TOK), z3),
            pl.BlockSpec((TOK, 16), lambda i: (0, 0)),
            pl.BlockSpec((TOK, 16), lambda i: (0, 0)),
            pl.BlockSpec((1, 4, TG), z3),
        ),
        out_shape=out_shapes,
        scratch_shapes=[
            pltpu.VMEM((1, E), jnp.float32),
            pltpu.VMEM((RNB, RTB), jnp.int32),
            pltpu.VMEM((RNB, RTB), jnp.int32),
            pltpu.VMEM((RNB, RTB), jnp.float32),
            pltpu.VMEM((RNB, RTB), jnp.float32),
            pltpu.VMEM((TOK, 1), jnp.float32),
        ],
    )(x2d, router_W, router_b.reshape(1, E))


def _gmm_body(tl_ref,
              xs_ref, w1_ref, b1_ref, w2_ref, b2_ref, out_ref):
    i = pl.program_id(0)
    lo = tl_ref[0, 2, i]
    hi = tl_ref[0, 3, i]
    x = xs_ref[...]
    h = jax.lax.dot_general(x, w1_ref[0], (((1,), (0,)), ((), ())),
                            preferred_element_type=jnp.float32) + b1_ref[0]
    h1 = h[:, :HIDDEN]
    h2 = h[:, HIDDEN:]
    act = h1 * jax.nn.sigmoid(h1) * h2
    y = jax.lax.dot_general(act, w2_ref[0], (((1,), (0,)), ((), ())),
                            preferred_element_type=jnp.float32) + b2_ref[0]
    ridx = jax.lax.broadcasted_iota(jnp.int32, (BT, 1), 0)
    mask = (ridx >= lo) & (ridx < hi)
    out_ref[...] = jnp.where(mask, y, out_ref[...])


def _run_gmm(xs, W1, b1, W2, b2, tiles):
    grid_spec = pltpu.PrefetchScalarGridSpec(
        num_scalar_prefetch=1,
        grid=(TG,),
        in_specs=[
            pl.BlockSpec((BT, DIM), lambda i, tl: (tl[0, 0, i], 0)),
            pl.BlockSpec((1, DIM, 2 * HIDDEN), lambda i, tl: (tl[0, 1, i], 0, 0)),
            pl.BlockSpec((1, 1, 2 * HIDDEN), lambda i, tl: (tl[0, 1, i], 0, 0)),
            pl.BlockSpec((1, HIDDEN, DIM), lambda i, tl: (tl[0, 1, i], 0, 0)),
            pl.BlockSpec((1, 1, DIM), lambda i, tl: (tl[0, 1, i], 0, 0)),
        ],
        out_specs=pl.BlockSpec((BT, DIM), lambda i, tl: (tl[0, 0, i], 0)),
    )
    return pl.pallas_call(
        _gmm_body,
        grid_spec=grid_spec,
        out_shape=jax.ShapeDtypeStruct((NA, DIM), jnp.float32),
    )(tiles,
      xs, W1, b1.reshape(E, 1, 2 * HIDDEN), W2, b2.reshape(E, 1, DIM))


_NW = 32
TPW = TOK // _NW
DCT = 128
CCT = 32


def _sc_mesh():
    return plsc.VectorSubcoreMesh(core_axis_name="c", subcore_axis_name="s",
                                  num_cores=2, num_subcores=16)


def _wid():
    return lax.axis_index("s") * 2 + lax.axis_index("c")


def _sc_dispatch(x2d, d1, d2):
    @functools.partial(
        pl.kernel,
        out_type=jax.ShapeDtypeStruct((NA, DIM), jnp.float32),
        mesh=_sc_mesh(),
        scratch_types=[
            pltpu.VMEM((DCT, DIM), jnp.float32),
            pltpu.VMEM((DCT,), jnp.int32),
            pltpu.VMEM((DCT,), jnp.int32),
            pltpu.SemaphoreType.DMA,
            pltpu.SemaphoreType.DMA,
        ],
    )
    def body(x_hbm, d1_hbm, d2_hbm, xs_hbm, rows_v, i1_v, i2_v, s1, s2):
        tbase = _wid() * TPW
        pltpu.sync_copy(x_hbm.at[pl.ds(tbase, DCT)], rows_v)
        pltpu.sync_copy(d1_hbm.at[pl.ds(tbase, DCT)], i1_v)
        pltpu.sync_copy(d2_hbm.at[pl.ds(tbase, DCT)], i2_v)
        c1 = pltpu.async_copy(rows_v, xs_hbm.at[i1_v], s1)
        c2 = pltpu.async_copy(rows_v, xs_hbm.at[i2_v], s2)
        c1.wait()
        c2.wait()

    return body(x2d, d1, d2)


def _sc_combine(ys, d1, d2, w1x, w2x):
    @functools.partial(
        pl.kernel,
        out_type=jax.ShapeDtypeStruct((TOK, DIM), jnp.float32),
        mesh=_sc_mesh(),
        scratch_types=[
            pltpu.VMEM((CCT, DIM), jnp.float32),
            pltpu.VMEM((CCT, DIM), jnp.float32),
            pltpu.VMEM((CCT, DIM), jnp.float32),
            pltpu.VMEM((CCT,), jnp.int32),
            pltpu.VMEM((CCT,), jnp.int32),
            pltpu.VMEM((CCT, 16), jnp.float32),
            pltpu.VMEM((CCT, 16), jnp.float32),
            pltpu.SemaphoreType.DMA,
            pltpu.SemaphoreType.DMA,
        ],
    )
    def body(ys_hbm, d1_hbm, d2_hbm, w1_hbm, w2_hbm, out_hbm,
             r1_v, r2_v, o_v, i1_v, i2_v, w1_v, w2_v, s1, s2):
        base = _wid() * TPW
        for ci in range(TPW // CCT):
            tbase = base + ci * CCT
            pltpu.sync_copy(d1_hbm.at[pl.ds(tbase, CCT)], i1_v)
            pltpu.sync_copy(d2_hbm.at[pl.ds(tbase, CCT)], i2_v)
            pltpu.sync_copy(w1_hbm.at[pl.ds(tbase, CCT)], w1_v)
            pltpu.sync_copy(w2_hbm.at[pl.ds(tbase, CCT)], w2_v)
            c1 = pltpu.async_copy(ys_hbm.at[i1_v], r1_v, s1)
            c2 = pltpu.async_copy(ys_hbm.at[i2_v], r2_v, s2)
            c1.wait()
            c2.wait()

            def tok(t, carry):
                a = w1_v[t, :]
                b = w2_v[t, :]
                for j in range(DIM // 16):
                    sl = pl.ds(j * 16, 16)
                    o_v[t, sl] = a * r1_v[t, sl] + b * r2_v[t, sl]
                return carry

            lax.fori_loop(0, CCT, tok, 0)
            pltpu.sync_copy(o_v, out_hbm.at[pl.ds(tbase, CCT)])

    return body(ys, d1, d2, w1x, w2x)


def kernel(x, router_W, router_b, W1, b1, W2, b2):
    x2d = x.reshape(TOK, DIM)
    d1, d2, w1x, w2x, tiles = _run_router(x2d, router_W, router_b)
    d1 = d1.reshape(TOK)
    d2 = d2.reshape(TOK)
    xs = _sc_dispatch(x2d, d1, d2)
    ys = _run_gmm(xs, W1, b1, W2, b2, tiles)
    out = _sc_combine(ys, d1, d2, w1x, w2x)
    return out.reshape(B, S, DIM)


# speedup vs baseline: 1.0121x; 1.0121x over previous
import functools

import jax
import jax.numpy as jnp
from jax import lax
from jax.experimental import pallas as pl
from jax.experimental.pallas import tpu as pltpu
from jax.experimental.pallas import tpu_sc as plsc

B, S, DIM = 2, 2048, 768
E, K, HIDDEN = 8, 2, 2048
TOK = B * S
NA = TOK * K

RTB = 256
RNB = TOK // RTB

BT = 512
NBG = NA // BT
TG = NBG + E - 1


def _router_body(x_ref, rw_ref, rb_ref,
                 d1_ref, d2_ref, w1x_ref, w2x_ref, tiles_ref,
                 acc_ref, se1_ref, se2_ref, sr1_ref, sr2_ref, sw1_ref):
    i = pl.program_id(0)

    @pl.when(i == 0)
    def _():
        acc_ref[...] = jnp.zeros_like(acc_ref)

    @pl.when(i < RNB)
    def _phase0():
        prev = acc_ref[...]
        x = x_ref[...]
        logits = jax.lax.dot_general(x, rw_ref[...], (((1,), (1,)), ((), ())),
                                     preferred_element_type=jnp.float32)
        logits = logits + rb_ref[...]

        eids = jax.lax.broadcasted_iota(jnp.int32, (RTB, E), 1)
        v1 = jnp.max(logits, axis=1, keepdims=True)
        i1 = jnp.min(jnp.where(logits == v1, eids, E), axis=1)
        masked = jnp.where(eids == i1[:, None], -jnp.inf, logits)
        v2 = jnp.max(masked, axis=1, keepdims=True)
        i2 = jnp.min(jnp.where(masked == v2, eids, E), axis=1)

        w1 = 1.0 / (1.0 + jnp.exp(v2[:, 0] - v1[:, 0]))

        a1 = (eids == i1[:, None]).astype(jnp.float32)
        a2 = (eids == i2[:, None]).astype(jnp.float32)
        s = a1 + a2
        tri = (jax.lax.broadcasted_iota(jnp.int32, (RTB, RTB), 0)
               > jax.lax.broadcasted_iota(jnp.int32, (RTB, RTB), 1)
               ).astype(jnp.float32)
        csum = jax.lax.dot_general(tri, s, (((1,), (0,)), ((), ())),
                                   precision=jax.lax.Precision.HIGHEST,
                                   preferred_element_type=jnp.float32)
        ctot = csum + prev
        r1 = jnp.sum(ctot * a1, axis=1)
        r2 = jnp.sum(ctot * a2, axis=1)

        se1_ref[pl.ds(i, 1), :] = i1[None, :]
        se2_ref[pl.ds(i, 1), :] = i2[None, :]
        sr1_ref[pl.ds(i, 1), :] = r1[None, :]
        sr2_ref[pl.ds(i, 1), :] = r2[None, :]
        sw1_ref[pl.ds(i * RTB, RTB), :] = w1[:, None]
        acc_ref[...] = prev + jnp.sum(s, axis=0, keepdims=True)

    @pl.when(i == RNB)
    def _finalize():
        def rowdot(a, b):
            return jax.lax.dot_general(a, b, (((1,), (0,)), ((), ())),
                                       precision=jax.lax.Precision.HIGHEST,
                                       preferred_element_type=jnp.float32)

        cnt = acc_ref[...]
        lt = (jax.lax.broadcasted_iota(jnp.int32, (E, E), 0)
              < jax.lax.broadcasted_iota(jnp.int32, (E, E), 1)).astype(jnp.float32)
        off_e = rowdot(cnt, lt)
        off_i = off_e + cnt

        for rb in range(RNB):
            e1b = se1_ref[pl.ds(rb, 1), :]
            e2b = se2_ref[pl.ds(rb, 1), :]
            eids = jax.lax.broadcasted_iota(jnp.int32, (E, RTB), 0)
            a1 = (eids == e1b).astype(jnp.float32)
            a2 = (eids == e2b).astype(jnp.float32)
            d1 = rowdot(off_e, a1) + sr1_ref[pl.ds(rb, 1), :]
            d2 = rowdot(off_e, a2) + sr2_ref[pl.ds(rb, 1), :]
            d1_ref[0, :, rb * RTB:(rb + 1) * RTB] = d1.astype(jnp.int32)
            d2_ref[0, :, rb * RTB:(rb + 1) * RTB] = d2.astype(jnp.int32)

        w1x_ref[...] = jnp.broadcast_to(sw1_ref[...], (TOK, 16))
        w2x_ref[...] = 1.0 - w1x_ref[...]

        inv_bt = 1.0 / BT
        nonempty = (cnt > 0).astype(jnp.float32)
        nb = (jnp.floor((off_i - 1.0) * inv_bt) - jnp.floor(off_e * inv_bt)
              + 1.0) * nonempty
        le = (jax.lax.broadcasted_iota(jnp.int32, (E, E), 0)
              <= jax.lax.broadcasted_iota(jnp.int32, (E, E), 1)).astype(jnp.float32)
        ts_i = rowdot(nb, le)
        ts_e = ts_i - nb
        total = jnp.max(ts_i, axis=1, keepdims=True)

        ident = (jax.lax.broadcasted_iota(jnp.int32, (E, E), 0)
                 == jax.lax.broadcasted_iota(jnp.int32, (E, E), 1)
                 ).astype(jnp.float32)
        ts_col = jax.lax.dot_general(ident, ts_i, (((1,), (1,)), ((), ())),
                                     precision=jax.lax.Precision.HIGHEST,
                                     preferred_element_type=jnp.float32)

        tids = jax.lax.broadcasted_iota(jnp.int32, (1, TG), 1).astype(jnp.float32)
        tc = jnp.minimum(tids, total - 1.0)
        cmp = (ts_col <= tc).astype(jnp.float32)
        et = jnp.sum(cmp, axis=0, keepdims=True)
        sel = (jax.lax.broadcasted_iota(jnp.int32, (E, TG), 0).astype(jnp.float32)
               == et).astype(jnp.float32)
        offe_t = rowdot(off_e, sel)
        offi_t = rowdot(off_i, sel)
        tse_t = rowdot(ts_e, sel)
        bt = jnp.floor(offe_t * inv_bt) + (tc - tse_t)
        lo = jnp.maximum(offe_t - bt * BT, 0.0)
        hi = jnp.minimum(offi_t - bt * BT, float(BT))
        tiles = jnp.concatenate([bt, et, lo, hi], axis=0)
        tiles_ref[...] = tiles.astype(jnp.int32)[None]


def _run_router(x2d, router_W, router_b):
    out_shapes = (
        jax.ShapeDtypeStruct((1, 1, TOK), jnp.int32),
        jax.ShapeDtypeStruct((1, 1, TOK), jnp.int32),
        jax.ShapeDtypeStruct((TOK, 16), jnp.float32),
        jax.ShapeDtypeStruct((TOK, 16), jnp.float32),
        jax.ShapeDtypeStruct((1, 4, TG), jnp.int32),
    )
    z3 = lambda i: (0, 0, 0)
    return pl.pallas_call(
        _router_body,
        grid=(RNB + 1,),
        in_specs=[
            pl.BlockSpec((RTB, DIM), lambda i: (jnp.minimum(i, RNB - 1), 0)),
            pl.BlockSpec((E, DIM), lambda i: (0, 0)),
            pl.BlockSpec((1, E), lambda i: (0, 0)),
        ],
        out_specs=(
            pl.BlockSpec((1, 1, TOK), z3), pl.BlockSpec((1, 1, TOK), z3),
            pl.BlockSpec((TOK, 16), lambda i: (0, 0)),
            pl.BlockSpec((TOK, 16), lambda i: (0, 0)),
            pl.BlockSpec((1, 4, TG), z3),
        ),
        out_shape=out_shapes,
        scratch_shapes=[
            pltpu.VMEM((1, E), jnp.float32),
            pltpu.VMEM((RNB, RTB), jnp.int32),
            pltpu.VMEM((RNB, RTB), jnp.int32),
            pltpu.VMEM((RNB, RTB), jnp.float32),
            pltpu.VMEM((RNB, RTB), jnp.float32),
            pltpu.VMEM((TOK, 1), jnp.float32),
        ],
    )(x2d, router_W, router_b.reshape(1, E))


def _gmm_body(tl_ref,
              xs_ref, w1_ref, b1_ref, w2_ref, b2_ref, out_ref):
    i = pl.program_id(0)
    lo = tl_ref[0, 2, i]
    hi = tl_ref[0, 3, i]
    x = xs_ref[...]
    h = jax.lax.dot_general(x, w1_ref[0], (((1,), (0,)), ((), ())),
                            preferred_element_type=jnp.float32) + b1_ref[0]
    h1 = h[:, :HIDDEN]
    h2 = h[:, HIDDEN:]
    act = h1 * jax.nn.sigmoid(h1) * h2
    y = jax.lax.dot_general(act, w2_ref[0], (((1,), (0,)), ((), ())),
                            preferred_element_type=jnp.float32) + b2_ref[0]
    ridx = jax.lax.broadcasted_iota(jnp.int32, (BT, 1), 0)
    mask = (ridx >= lo) & (ridx < hi)
    out_ref[...] = jnp.where(mask, y, out_ref[...])


def _run_gmm(xs, W1, b1, W2, b2, tiles):
    grid_spec = pltpu.PrefetchScalarGridSpec(
        num_scalar_prefetch=1,
        grid=(TG,),
        in_specs=[
            pl.BlockSpec((BT, DIM), lambda i, tl: (tl[0, 0, i], 0)),
            pl.BlockSpec((1, DIM, 2 * HIDDEN), lambda i, tl: (tl[0, 1, i], 0, 0)),
            pl.BlockSpec((1, 1, 2 * HIDDEN), lambda i, tl: (tl[0, 1, i], 0, 0)),
            pl.BlockSpec((1, HIDDEN, DIM), lambda i, tl: (tl[0, 1, i], 0, 0)),
            pl.BlockSpec((1, 1, DIM), lambda i, tl: (tl[0, 1, i], 0, 0)),
        ],
        out_specs=pl.BlockSpec((BT, DIM), lambda i, tl: (tl[0, 0, i], 0)),
    )
    return pl.pallas_call(
        _gmm_body,
        grid_spec=grid_spec,
        out_shape=jax.ShapeDtypeStruct((NA, DIM), jnp.float32),
    )(tiles,
      xs, W1, b1.reshape(E, 1, 2 * HIDDEN), W2, b2.reshape(E, 1, DIM))


_NW = 32
TPW = TOK // _NW
DCT = 128
CCT = 32


def _sc_mesh():
    return plsc.VectorSubcoreMesh(core_axis_name="c", subcore_axis_name="s",
                                  num_cores=2, num_subcores=16)


def _wid():
    return lax.axis_index("s") * 2 + lax.axis_index("c")


def _sc_dispatch(x2d, d1, d2):
    @functools.partial(
        pl.kernel,
        out_type=jax.ShapeDtypeStruct((NA, DIM), jnp.float32),
        mesh=_sc_mesh(),
        scratch_types=[
            pltpu.VMEM((DCT, DIM), jnp.float32),
            pltpu.VMEM((DCT,), jnp.int32),
            pltpu.VMEM((DCT,), jnp.int32),
            pltpu.SemaphoreType.DMA,
            pltpu.SemaphoreType.DMA,
        ],
    )
    def body(x_hbm, d1_hbm, d2_hbm, xs_hbm, rows_v, i1_v, i2_v, s1, s2):
        tbase = _wid() * TPW
        pltpu.sync_copy(x_hbm.at[pl.ds(tbase, DCT)], rows_v)
        pltpu.sync_copy(d1_hbm.at[pl.ds(tbase, DCT)], i1_v)
        pltpu.sync_copy(d2_hbm.at[pl.ds(tbase, DCT)], i2_v)
        c1 = pltpu.async_copy(rows_v, xs_hbm.at[i1_v], s1)
        c2 = pltpu.async_copy(rows_v, xs_hbm.at[i2_v], s2)
        c1.wait()
        c2.wait()

    return body(x2d, d1, d2)


def _sc_combine(ys, d1, d2, w1x, w2x):
    @functools.partial(
        pl.kernel,
        out_type=jax.ShapeDtypeStruct((TOK, DIM), jnp.float32),
        mesh=_sc_mesh(),
        scratch_types=[
            pltpu.VMEM((CCT, DIM), jnp.float32),
            pltpu.VMEM((CCT, DIM), jnp.float32),
            pltpu.VMEM((CCT, DIM), jnp.float32),
            pltpu.VMEM((CCT,), jnp.int32),
            pltpu.VMEM((CCT,), jnp.int32),
            pltpu.VMEM((CCT, 16), jnp.float32),
            pltpu.VMEM((CCT, 16), jnp.float32),
            pltpu.SemaphoreType.DMA,
            pltpu.SemaphoreType.DMA,
        ],
    )
    def body(ys_hbm, d1_hbm, d2_hbm, w1_hbm, w2_hbm, out_hbm,
             r1_v, r2_v, o_v, i1_v, i2_v, w1_v, w2_v, s1, s2):
        base = _wid() * TPW
        for ci in range(TPW // CCT):
            tbase = base + ci * CCT
            pltpu.sync_copy(d1_hbm.at[pl.ds(tbase, CCT)], i1_v)
            pltpu.sync_copy(d2_hbm.at[pl.ds(tbase, CCT)], i2_v)
            pltpu.sync_copy(w1_hbm.at[pl.ds(tbase, CCT)], w1_v)
            pltpu.sync_copy(w2_hbm.at[pl.ds(tbase, CCT)], w2_v)
            c1 = pltpu.async_copy(ys_hbm.at[i1_v], r1_v, s1)
            c2 = pltpu.async_copy(ys_hbm.at[i2_v], r2_v, s2)
            c1.wait()
            c2.wait()

            def tok(t, carry):
                a = w1_v[t, :]
                b = w2_v[t, :]
                for j in range(DIM // 16):
                    sl = pl.ds(j * 16, 16)
                    o_v[t, sl] = a * r1_v[t, sl] + b * r2_v[t, sl]
                return carry

            lax.fori_loop(0, CCT, tok, 0)
            pltpu.sync_copy(o_v, out_hbm.at[pl.ds(tbase, CCT)])

    return body(ys, d1, d2, w1x, w2x)


def kernel(x, router_W, router_b, W1, b1, W2, b2):
    x2d = x.reshape(TOK, DIM)
    d1, d2, w1x, w2x, tiles = _run_router(x2d, router_W, router_b)
    d1 = d1.reshape(TOK)
    d2 = d2.reshape(TOK)
    xs = _sc_dispatch(x2d, d1, d2)
    ys = _run_gmm(xs, W1, b1, W2, b2, tiles)
    out = _sc_combine(ys, d1, d2, w1x, w2x)
    return out.reshape(B, S, DIM)


# speedup vs baseline: 1.0190x; 1.0069x over previous
import functools

import jax
import jax.numpy as jnp
from jax import lax
from jax.experimental import pallas as pl
from jax.experimental.pallas import tpu as pltpu
from jax.experimental.pallas import tpu_sc as plsc

B, S, DIM = 2, 2048, 768
E, K, HIDDEN = 8, 2, 2048
TOK = B * S
NA = TOK * K

RTB = 512
RNB = TOK // RTB

BT = 512
NBG = NA // BT
TG = NBG + E - 1


def _router_body(x_ref, rw_ref, rb_ref,
                 d1_ref, d2_ref, w1x_ref, w2x_ref, tiles_ref,
                 acc_ref, se1_ref, se2_ref, sr1_ref, sr2_ref, sw1_ref):
    i = pl.program_id(0)

    @pl.when(i == 0)
    def _():
        acc_ref[...] = jnp.zeros_like(acc_ref)

    @pl.when(i < RNB)
    def _phase0():
        prev = acc_ref[...]
        x = x_ref[...]
        logits = jax.lax.dot_general(x, rw_ref[...], (((1,), (1,)), ((), ())),
                                     preferred_element_type=jnp.float32)
        logits = logits + rb_ref[...]

        eids = jax.lax.broadcasted_iota(jnp.int32, (RTB, E), 1)
        v1 = jnp.max(logits, axis=1, keepdims=True)
        i1 = jnp.min(jnp.where(logits == v1, eids, E), axis=1)
        masked = jnp.where(eids == i1[:, None], -jnp.inf, logits)
        v2 = jnp.max(masked, axis=1, keepdims=True)
        i2 = jnp.min(jnp.where(masked == v2, eids, E), axis=1)

        w1 = 1.0 / (1.0 + jnp.exp(v2[:, 0] - v1[:, 0]))

        a1 = (eids == i1[:, None]).astype(jnp.float32)
        a2 = (eids == i2[:, None]).astype(jnp.float32)
        s = a1 + a2
        tri = (jax.lax.broadcasted_iota(jnp.int32, (RTB, RTB), 0)
               > jax.lax.broadcasted_iota(jnp.int32, (RTB, RTB), 1)
               ).astype(jnp.float32)
        csum = jax.lax.dot_general(tri, s, (((1,), (0,)), ((), ())),
                                   precision=jax.lax.Precision.HIGHEST,
                                   preferred_element_type=jnp.float32)
        ctot = csum + prev
        r1 = jnp.sum(ctot * a1, axis=1)
        r2 = jnp.sum(ctot * a2, axis=1)

        se1_ref[pl.ds(i, 1), :] = i1[None, :]
        se2_ref[pl.ds(i, 1), :] = i2[None, :]
        sr1_ref[pl.ds(i, 1), :] = r1[None, :]
        sr2_ref[pl.ds(i, 1), :] = r2[None, :]
        sw1_ref[pl.ds(i * RTB, RTB), :] = w1[:, None]
        acc_ref[...] = prev + jnp.sum(s, axis=0, keepdims=True)

    @pl.when(i == RNB)
    def _finalize():
        def rowdot(a, b):
            return jax.lax.dot_general(a, b, (((1,), (0,)), ((), ())),
                                       precision=jax.lax.Precision.HIGHEST,
                                       preferred_element_type=jnp.float32)

        cnt = acc_ref[...]
        lt = (jax.lax.broadcasted_iota(jnp.int32, (E, E), 0)
              < jax.lax.broadcasted_iota(jnp.int32, (E, E), 1)).astype(jnp.float32)
        off_e = rowdot(cnt, lt)
        off_i = off_e + cnt

        for rb in range(RNB):
            e1b = se1_ref[pl.ds(rb, 1), :]
            e2b = se2_ref[pl.ds(rb, 1), :]
            eids = jax.lax.broadcasted_iota(jnp.int32, (E, RTB), 0)
            a1 = (eids == e1b).astype(jnp.float32)
            a2 = (eids == e2b).astype(jnp.float32)
            d1 = rowdot(off_e, a1) + sr1_ref[pl.ds(rb, 1), :]
            d2 = rowdot(off_e, a2) + sr2_ref[pl.ds(rb, 1), :]
            d1_ref[0, :, rb * RTB:(rb + 1) * RTB] = d1.astype(jnp.int32)
            d2_ref[0, :, rb * RTB:(rb + 1) * RTB] = d2.astype(jnp.int32)

        w1x_ref[...] = jnp.broadcast_to(sw1_ref[...], (TOK, 16))
        w2x_ref[...] = 1.0 - w1x_ref[...]

        inv_bt = 1.0 / BT
        nonempty = (cnt > 0).astype(jnp.float32)
        nb = (jnp.floor((off_i - 1.0) * inv_bt) - jnp.floor(off_e * inv_bt)
              + 1.0) * nonempty
        le = (jax.lax.broadcasted_iota(jnp.int32, (E, E), 0)
              <= jax.lax.broadcasted_iota(jnp.int32, (E, E), 1)).astype(jnp.float32)
        ts_i = rowdot(nb, le)
        ts_e = ts_i - nb
        total = jnp.max(ts_i, axis=1, keepdims=True)

        ident = (jax.lax.broadcasted_iota(jnp.int32, (E, E), 0)
                 == jax.lax.broadcasted_iota(jnp.int32, (E, E), 1)
                 ).astype(jnp.float32)
        ts_col = jax.lax.dot_general(ident, ts_i, (((1,), (1,)), ((), ())),
                                     precision=jax.lax.Precision.HIGHEST,
                                     preferred_element_type=jnp.float32)

        tids = jax.lax.broadcasted_iota(jnp.int32, (1, TG), 1).astype(jnp.float32)
        tc = jnp.minimum(tids, total - 1.0)
        cmp = (ts_col <= tc).astype(jnp.float32)
        et = jnp.sum(cmp, axis=0, keepdims=True)
        sel = (jax.lax.broadcasted_iota(jnp.int32, (E, TG), 0).astype(jnp.float32)
               == et).astype(jnp.float32)
        offe_t = rowdot(off_e, sel)
        offi_t = rowdot(off_i, sel)
        tse_t = rowdot(ts_e, sel)
        bt = jnp.floor(offe_t * inv_bt) + (tc - tse_t)
        lo = jnp.maximum(offe_t - bt * BT, 0.0)
        hi = jnp.minimum(offi_t - bt * BT, float(BT))
        tiles = jnp.concatenate([bt, et, lo, hi], axis=0)
        tiles_ref[...] = tiles.astype(jnp.int32)[None]


def _run_router(x2d, router_W, router_b):
    out_shapes = (
        jax.ShapeDtypeStruct((1, 1, TOK), jnp.int32),
        jax.ShapeDtypeStruct((1, 1, TOK), jnp.int32),
        jax.ShapeDtypeStruct((TOK, 16), jnp.float32),
        jax.ShapeDtypeStruct((TOK, 16), jnp.float32),
        jax.ShapeDtypeStruct((1, 4, TG), jnp.int32),
    )
    z3 = lambda i: (0, 0, 0)
    return pl.pallas_call(
        _router_body,
        grid=(RNB + 1,),
        in_specs=[
            pl.BlockSpec((RTB, DIM), lambda i: (jnp.minimum(i, RNB - 1), 0)),
            pl.BlockSpec((E, DIM), lambda i: (0, 0)),
            pl.BlockSpec((1, E), lambda i: (0, 0)),
        ],
        out_specs=(
            pl.BlockSpec((1, 1, TOK), z3), pl.BlockSpec((1, 1, TOK), z3),
            pl.BlockSpec((TOK, 16), lambda i: (0, 0)),
            pl.BlockSpec((TOK, 16), lambda i: (0, 0)),
            pl.BlockSpec((1, 4, TG), z3),
        ),
        out_shape=out_shapes,
        scratch_shapes=[
            pltpu.VMEM((1, E), jnp.float32),
            pltpu.VMEM((RNB, RTB), jnp.int32),
            pltpu.VMEM((RNB, RTB), jnp.int32),
            pltpu.VMEM((RNB, RTB), jnp.float32),
            pltpu.VMEM((RNB, RTB), jnp.float32),
            pltpu.VMEM((TOK, 1), jnp.float32),
        ],
    )(x2d, router_W, router_b.reshape(1, E))


def _gmm_body(tl_ref,
              xs_ref, w1_ref, b1_ref, w2_ref, b2_ref, out_ref):
    i = pl.program_id(0)
    lo = tl_ref[0, 2, i]
    hi = tl_ref[0, 3, i]
    x = xs_ref[...]
    h = jax.lax.dot_general(x, w1_ref[0], (((1,), (0,)), ((), ())),
                            preferred_element_type=jnp.float32) + b1_ref[0]
    h1 = h[:, :HIDDEN]
    h2 = h[:, HIDDEN:]
    act = h1 * jax.nn.sigmoid(h1) * h2
    y = jax.lax.dot_general(act, w2_ref[0], (((1,), (0,)), ((), ())),
                            preferred_element_type=jnp.float32) + b2_ref[0]
    ridx = jax.lax.broadcasted_iota(jnp.int32, (BT, 1), 0)
    mask = (ridx >= lo) & (ridx < hi)
    out_ref[...] = jnp.where(mask, y, out_ref[...])


def _run_gmm(xs, W1, b1, W2, b2, tiles):
    grid_spec = pltpu.PrefetchScalarGridSpec(
        num_scalar_prefetch=1,
        grid=(TG,),
        in_specs=[
            pl.BlockSpec((BT, DIM), lambda i, tl: (tl[0, 0, i], 0)),
            pl.BlockSpec((1, DIM, 2 * HIDDEN), lambda i, tl: (tl[0, 1, i], 0, 0)),
            pl.BlockSpec((1, 1, 2 * HIDDEN), lambda i, tl: (tl[0, 1, i], 0, 0)),
            pl.BlockSpec((1, HIDDEN, DIM), lambda i, tl: (tl[0, 1, i], 0, 0)),
            pl.BlockSpec((1, 1, DIM), lambda i, tl: (tl[0, 1, i], 0, 0)),
        ],
        out_specs=pl.BlockSpec((BT, DIM), lambda i, tl: (tl[0, 0, i], 0)),
    )
    return pl.pallas_call(
        _gmm_body,
        grid_spec=grid_spec,
        out_shape=jax.ShapeDtypeStruct((NA, DIM), jnp.float32),
    )(tiles,
      xs, W1, b1.reshape(E, 1, 2 * HIDDEN), W2, b2.reshape(E, 1, DIM))


_NW = 32
TPW = TOK // _NW
DCT = 128
CCT = 32


def _sc_mesh():
    return plsc.VectorSubcoreMesh(core_axis_name="c", subcore_axis_name="s",
                                  num_cores=2, num_subcores=16)


def _wid():
    return lax.axis_index("s") * 2 + lax.axis_index("c")


def _sc_dispatch(x2d, d1, d2):
    @functools.partial(
        pl.kernel,
        out_type=jax.ShapeDtypeStruct((NA, DIM), jnp.float32),
        mesh=_sc_mesh(),
        scratch_types=[
            pltpu.VMEM((DCT, DIM), jnp.float32),
            pltpu.VMEM((DCT,), jnp.int32),
            pltpu.VMEM((DCT,), jnp.int32),
            pltpu.SemaphoreType.DMA,
            pltpu.SemaphoreType.DMA,
        ],
    )
    def body(x_hbm, d1_hbm, d2_hbm, xs_hbm, rows_v, i1_v, i2_v, s1, s2):
        tbase = _wid() * TPW
        pltpu.sync_copy(x_hbm.at[pl.ds(tbase, DCT)], rows_v)
        pltpu.sync_copy(d1_hbm.at[pl.ds(tbase, DCT)], i1_v)
        pltpu.sync_copy(d2_hbm.at[pl.ds(tbase, DCT)], i2_v)
        c1 = pltpu.async_copy(rows_v, xs_hbm.at[i1_v], s1)
        c2 = pltpu.async_copy(rows_v, xs_hbm.at[i2_v], s2)
        c1.wait()
        c2.wait()

    return body(x2d, d1, d2)


def _sc_combine(ys, d1, d2, w1x, w2x):
    @functools.partial(
        pl.kernel,
        out_type=jax.ShapeDtypeStruct((TOK, DIM), jnp.float32),
        mesh=_sc_mesh(),
        scratch_types=[
            pltpu.VMEM((CCT, DIM), jnp.float32),
            pltpu.VMEM((CCT, DIM), jnp.float32),
            pltpu.VMEM((CCT, DIM), jnp.float32),
            pltpu.VMEM((CCT,), jnp.int32),
            pltpu.VMEM((CCT,), jnp.int32),
            pltpu.VMEM((CCT, 16), jnp.float32),
            pltpu.VMEM((CCT, 16), jnp.float32),
            pltpu.SemaphoreType.DMA,
            pltpu.SemaphoreType.DMA,
        ],
    )
    def body(ys_hbm, d1_hbm, d2_hbm, w1_hbm, w2_hbm, out_hbm,
             r1_v, r2_v, o_v, i1_v, i2_v, w1_v, w2_v, s1, s2):
        base = _wid() * TPW
        for ci in range(TPW // CCT):
            tbase = base + ci * CCT
            pltpu.sync_copy(d1_hbm.at[pl.ds(tbase, CCT)], i1_v)
            pltpu.sync_copy(d2_hbm.at[pl.ds(tbase, CCT)], i2_v)
            pltpu.sync_copy(w1_hbm.at[pl.ds(tbase, CCT)], w1_v)
            pltpu.sync_copy(w2_hbm.at[pl.ds(tbase, CCT)], w2_v)
            c1 = pltpu.async_copy(ys_hbm.at[i1_v], r1_v, s1)
            c2 = pltpu.async_copy(ys_hbm.at[i2_v], r2_v, s2)
            c1.wait()
            c2.wait()

            def tok(t, carry):
                a = w1_v[t, :]
                b = w2_v[t, :]
                for j in range(DIM // 16):
                    sl = pl.ds(j * 16, 16)
                    o_v[t, sl] = a * r1_v[t, sl] + b * r2_v[t, sl]
                return carry

            lax.fori_loop(0, CCT, tok, 0)
            pltpu.sync_copy(o_v, out_hbm.at[pl.ds(tbase, CCT)])

    return body(ys, d1, d2, w1x, w2x)


def kernel(x, router_W, router_b, W1, b1, W2, b2):
    x2d = x.reshape(TOK, DIM)
    d1, d2, w1x, w2x, tiles = _run_router(x2d, router_W, router_b)
    d1 = d1.reshape(TOK)
    d2 = d2.reshape(TOK)
    xs = _sc_dispatch(x2d, d1, d2)
    ys = _run_gmm(xs, W1, b1, W2, b2, tiles)
    out = _sc_combine(ys, d1, d2, w1x, w2x)
    return out.reshape(B, S, DIM)
